# Initial kernel scaffold; baseline (speedup 1.0000x reference)
import jax
import jax.numpy as jnp
from jax.experimental import pallas as pl


def kernel(x, coords, W_fe1, b_fe1, W_fe2, b_fe2, g_n1, b_n1, Wq, Wk, Wv, Wo, ln_g, ln_b, W_in, W_out, A_log, w_dt, dt_bias, W_B, W_C, D_skip, g_f, b_f):
    raise NotImplementedError("write your pallas kernel here")


# speedup vs baseline: 58.2311x; 58.2311x over previous
import functools
import jax
import jax.numpy as jnp
from jax.experimental import pallas as pl
from jax.experimental.pallas import tpu as pltpu

N = 16384
H = 128
NH = 8
BLOCK = 128
DS = 4
C = 128
L = N // C
TS = 16
RB = 2048
AB = 256


def _ln_rows(x, g, b):
    m = x.mean(-1, keepdims=True)
    v = ((x - m) ** 2).mean(-1, keepdims=True)
    return (x - m) / jnp.sqrt(v + 1e-5) * g + b


def _attn_body(xs_ref, wf1_ref, bf1_ref, wf2_ref, bf2_ref, gn_ref, bn_ref,
               wq_ref, wk_ref, wv_ref, wo_ref, out_ref):
    x = xs_ref[...]
    h = jnp.maximum(jnp.dot(x, wf1_ref[...], preferred_element_type=jnp.float32)
                    + bf1_ref[...], 0.0)
    h = jnp.dot(h, wf2_ref[...], preferred_element_type=jnp.float32) + bf2_ref[...]
    xn = _ln_rows(h, gn_ref[...], bn_ref[...])
    q = jnp.dot(xn, wq_ref[...], preferred_element_type=jnp.float32)
    k = jnp.dot(xn, wk_ref[...], preferred_element_type=jnp.float32)
    v = jnp.dot(xn, wv_ref[...], preferred_element_type=jnp.float32)
    scale = 1.0 / jnp.sqrt(jnp.float32(H))
    obuf = []
    for b2 in range(AB // BLOCK):
        r0 = b2 * BLOCK
        orow = []
        for hh in range(NH):
            c0 = hh * H
            qh = q[r0:r0 + BLOCK, c0:c0 + H]
            kh = k[r0:r0 + BLOCK, c0:c0 + H]
            vh = v[r0:r0 + BLOCK, c0:c0 + H]
            s = jax.lax.dot_general(qh, kh, (((1,), (1,)), ((), ())),
                                    preferred_element_type=jnp.float32) * scale
            m = jnp.max(s, axis=-1, keepdims=True)
            p = jnp.exp(s - m)
            denom = jnp.sum(p, axis=-1, keepdims=True)
            attn = p / denom
            orow.append(jnp.dot(attn, vh, preferred_element_type=jnp.float32))
        obuf.append(jnp.concatenate(orow, axis=1))
    o = jnp.concatenate(obuf, axis=0)
    out_ref[...] = jnp.dot(o, wo_ref[...], preferred_element_type=jnp.float32)


def _attention(xs, wf1, bf1, wf2, bf2, gn, bn, wq, wk, wv, wo):
    grid = (N // AB,)
    full = lambda shape: pl.BlockSpec(shape, lambda i: (0, 0))
    return pl.pallas_call(
        _attn_body,
        grid=grid,
        in_specs=[
            pl.BlockSpec((AB, 8), lambda i: (i, 0)),
            full(wf1.shape), full(bf1.shape), full(wf2.shape), full(bf2.shape),
            full(gn.shape), full(bn.shape),
            full(wq.shape), full(wk.shape), full(wv.shape), full(wo.shape),
        ],
        out_specs=pl.BlockSpec((AB, H), lambda i: (i, 0)),
        out_shape=jax.ShapeDtypeStruct((N, H), jnp.float32),
    )(xs, wf1, bf1, wf2, bf2, gn, bn, wq, wk, wv, wo)


def _prep_body(r_ref, lng_ref, lnb_ref, win_ref, wdt_ref, dtb_ref, wbc_ref,
               dt_ref, u_ref, bc_ref, xi_ref, z_ref):
    r = r_ref[...]
    hn = _ln_rows(r, lng_ref[...], lnb_ref[...])
    xz = jnp.dot(hn, win_ref[...], preferred_element_type=jnp.float32)
    xi = xz[:, :H]
    z = xz[:, H:]
    pre = xi * wdt_ref[...] + dtb_ref[...]
    dt = jnp.maximum(pre, 0.0) + jnp.log1p(jnp.exp(-jnp.abs(pre)))
    dt_ref[...] = dt
    u_ref[...] = dt * xi
    bc_ref[...] = jnp.dot(xi, wbc_ref[...], preferred_element_type=jnp.float32)
    xi_ref[...] = xi
    z_ref[...] = z


def _prep(r, lng, lnb, win, wdt, dtb, wbc):
    grid = (N // RB,)
    full = lambda shape: pl.BlockSpec(shape, lambda i: (0, 0))
    return pl.pallas_call(
        _prep_body,
        grid=grid,
        in_specs=[
            pl.BlockSpec((RB, H), lambda i: (i, 0)),
            full(lng.shape), full(lnb.shape), full(win.shape),
            full(wdt.shape), full(dtb.shape), full(wbc.shape),
        ],
        out_specs=[
            pl.BlockSpec((RB, H), lambda i: (i, 0)),
            pl.BlockSpec((RB, H), lambda i: (i, 0)),
            pl.BlockSpec((RB, 8), lambda i: (i, 0)),
            pl.BlockSpec((RB, H), lambda i: (i, 0)),
            pl.BlockSpec((RB, H), lambda i: (i, 0)),
        ],
        out_shape=[
            jax.ShapeDtypeStruct((N, H), jnp.float32),
            jax.ShapeDtypeStruct((N, H), jnp.float32),
            jax.ShapeDtypeStruct((N, 8), jnp.float32),
            jax.ShapeDtypeStruct((N, H), jnp.float32),
            jax.ShapeDtypeStruct((N, H), jnp.float32),
        ],
    )(r, lng, lnb, win, wdt, dtb, wbc)


def _pass1_body(dt_ref, u_ref, bc_ref, an_ref, hend_ref, s_ref, h4, ssum):
    g = pl.program_id(0)

    @pl.when(g == 0)
    def _():
        h4[...] = jnp.zeros((DS, C, H), jnp.float32)
        ssum[...] = jnp.zeros((C, H), jnp.float32)

    for tt in range(TS):
        dt_t = dt_ref[:, tt, :]
        u_t = u_ref[:, tt, :]
        ssum[...] = ssum[...] + dt_t
        for j in range(DS):
            a = jnp.exp(dt_t * an_ref[j:j + 1, :])
            h4[j] = a * h4[j] + u_t * bc_ref[:, tt, j:j + 1]

    @pl.when(g == (L // TS) - 1)
    def _():
        hend_ref[...] = h4[...]
        s_ref[...] = ssum[...]


def _scan_pass1(dt3, u3, bc3, an):
    grid = (L // TS,)
    return pl.pallas_call(
        _pass1_body,
        grid=grid,
        in_specs=[
            pl.BlockSpec((C, TS, H), lambda g: (0, g, 0)),
            pl.BlockSpec((C, TS, H), lambda g: (0, g, 0)),
            pl.BlockSpec((C, TS, 8), lambda g: (0, g, 0)),
            pl.BlockSpec((DS, H), lambda g: (0, 0)),
        ],
        out_specs=[
            pl.BlockSpec((DS, C, H), lambda g: (0, 0, 0)),
            pl.BlockSpec((C, H), lambda g: (0, 0)),
        ],
        out_shape=[
            jax.ShapeDtypeStruct((DS, C, H), jnp.float32),
            jax.ShapeDtypeStruct((C, H), jnp.float32),
        ],
        scratch_shapes=[
            pltpu.VMEM((DS, C, H), jnp.float32),
            pltpu.VMEM((C, H), jnp.float32),
        ],
    )(dt3, u3, bc3, an)


def _shift_rows(x, s, fill):
    pad = jnp.full((s, x.shape[1]), fill, x.dtype)
    return jnp.concatenate([pad, x[:-s]], axis=0)


def _pass2_body(dt_ref, u_ref, bc_ref, an_ref, hend_ref, s_ref, ys_ref, h4):
    g = pl.program_id(0)

    @pl.when(g == 0)
    def _():
        stot = s_ref[...]
        for j in range(DS):
            e = jnp.exp(stot * an_ref[j:j + 1, :])
            hx = hend_ref[j]
            ep = e
            k = 1
            while k < C:
                hx = hx + ep * _shift_rows(hx, k, 0.0)
                ep = ep * _shift_rows(ep, k, 1.0)
                k *= 2
            h4[j] = _shift_rows(hx, 1, 0.0)

    for tt in range(TS):
        dt_t = dt_ref[:, tt, :]
        u_t = u_ref[:, tt, :]
        y = jnp.zeros((C, H), jnp.float32)
        for j in range(DS):
            a = jnp.exp(dt_t * an_ref[j:j + 1, :])
            hj = a * h4[j] + u_t * bc_ref[:, tt, j:j + 1]
            h4[j] = hj
            y = y + hj * bc_ref[:, tt, 4 + j:5 + j]
        ys_ref[:, tt, :] = y


def _scan_pass2(dt3, u3, bc3, an, hend, stot):
    grid = (L // TS,)
    return pl.pallas_call(
        _pass2_body,
        grid=grid,
        in_specs=[
            pl.BlockSpec((C, TS, H), lambda g: (0, g, 0)),
            pl.BlockSpec((C, TS, H), lambda g: (0, g, 0)),
            pl.BlockSpec((C, TS, 8), lambda g: (0, g, 0)),
            pl.BlockSpec((DS, H), lambda g: (0, 0)),
            pl.BlockSpec((DS, C, H), lambda g: (0, 0, 0)),
            pl.BlockSpec((C, H), lambda g: (0, 0)),
        ],
        out_specs=pl.BlockSpec((C, TS, H), lambda g: (0, g, 0)),
        out_shape=jax.ShapeDtypeStruct((C, L, H), jnp.float32),
        scratch_shapes=[pltpu.VMEM((DS, C, H), jnp.float32)],
    )(dt3, u3, bc3, an, hend, stot)


def _post_body(final, ys_ref, xi_ref, z_ref, r_ref, d_ref, wout_ref,
               gf_ref, bf_ref, out_ref):
    ys = ys_ref[...]
    xi = xi_ref[...]
    z = z_ref[...]
    y = (ys + d_ref[...] * xi) * (z / (1.0 + jnp.exp(-z)))
    hid = jnp.dot(y, wout_ref[...], preferred_element_type=jnp.float32)
    r = r_ref[...] + hid
    if final:
        r = _ln_rows(r, gf_ref[...], bf_ref[...])
    out_ref[...] = r


def _post(ys, xi, z, r, d, wout, gf, bf, final):
    grid = (N // RB,)
    full = lambda shape: pl.BlockSpec(shape, lambda i: (0, 0))
    return pl.pallas_call(
        functools.partial(_post_body, final),
        grid=grid,
        in_specs=[
            pl.BlockSpec((RB, H), lambda i: (i, 0)),
            pl.BlockSpec((RB, H), lambda i: (i, 0)),
            pl.BlockSpec((RB, H), lambda i: (i, 0)),
            pl.BlockSpec((RB, H), lambda i: (i, 0)),
            full(d.shape), full(wout.shape), full(gf.shape), full(bf.shape),
        ],
        out_specs=pl.BlockSpec((RB, H), lambda i: (i, 0)),
        out_shape=jax.ShapeDtypeStruct((N, H), jnp.float32),
    )(ys, xi, z, r, d, wout, gf, bf)


def kernel(x, coords, W_fe1, b_fe1, W_fe2, b_fe2, g_n1, b_n1, Wq, Wk, Wv, Wo,
           ln_g, ln_b, W_in, W_out, A_log, w_dt, dt_bias, W_B, W_C, D_skip,
           g_f, b_f):
    order = jnp.argsort(coords[:, 0])
    inv = jnp.argsort(order)

    xpad = jnp.pad(x, ((0, 0), (0, 2)))
    wf1 = jnp.pad(W_fe1, ((0, 2), (0, 0)))
    xs = jnp.take(xpad, order, axis=0)

    row = lambda a: a.reshape(1, -1)
    hidden_sorted = _attention(
        xs, wf1, row(b_fe1), W_fe2, row(b_fe2), row(g_n1), row(b_n1),
        Wq, Wk, Wv, Wo)
    r = jnp.take(hidden_sorted, inv, axis=0)

    for l in range(2):
        an = -jnp.exp(A_log[l]).T
        wbc = jnp.concatenate([W_B[l], W_C[l]], axis=1)
        dt, u, bc, xi, z = _prep(r, row(ln_g[l]), row(ln_b[l]), W_in[l],
                                 row(w_dt[l]), row(dt_bias[l]), wbc)
        dt3 = dt.reshape(C, L, H)
        u3 = u.reshape(C, L, H)
        bc3 = bc.reshape(C, L, 8)
        hend, stot = _scan_pass1(dt3, u3, bc3, an)
        ys = _scan_pass2(dt3, u3, bc3, an, hend, stot).reshape(N, H)
        r = _post(ys, xi, z, r, row(D_skip[l]), W_out[l],
                  row(g_f), row(b_f), final=(l == 1))
    return r
